# Initial kernel scaffold; baseline (speedup 1.0000x reference)
import jax
import jax.numpy as jnp
from jax.experimental import pallas as pl


def kernel(x, edge_index, batch, W0, b0, W1, b1, W2, b2, W3, b3, W4, b4, lin_W, lin_b):
    raise NotImplementedError("write your pallas kernel here")


# speedup vs baseline: 16.9052x; 16.9052x over previous
import functools

import jax
import jax.numpy as jnp
from jax import lax
from jax.experimental import pallas as pl
from jax.experimental.pallas import tpu as pltpu
from jax.experimental.pallas import tpu_sc as plsc

NC = 2
NS = 16
NW = NC * NS
CH = 80
HW = 64
DEGW = 16
NUM_GRAPHS = 16


def _sc_mesh():
    return plsc.VectorSubcoreMesh(
        core_axis_name="c", subcore_axis_name="s", num_cores=NC, num_subcores=NS
    )


@functools.lru_cache(maxsize=None)
def _edge_scatter_kernel(n_nodes, n_edges):
    per_s = n_edges // NS
    assert per_s * NS == n_edges
    n_chunks = per_s // CH
    assert n_chunks * CH == per_s and n_chunks % 2 == 0
    n_rchunks = n_nodes // CH
    assert n_rchunks * CH == n_nodes

    @functools.partial(
        pl.kernel,
        out_type=jax.ShapeDtypeStruct((NC, n_nodes, HW), jnp.float32),
        mesh=_sc_mesh(),
        compiler_params=pltpu.CompilerParams(use_tc_tiling_on_sc=False),
        scratch_types=[
            pltpu.VMEM((n_chunks, CH), jnp.int32),
            pltpu.VMEM((n_chunks, CH), jnp.int32),
            pltpu.VMEM((CH, HW), jnp.float32),
            pltpu.VMEM((CH, HW), jnp.float32),
            pltpu.SemaphoreType.DMA,
            pltpu.SemaphoreType.DMA,
            pltpu.VMEM_SHARED((n_nodes, HW), jnp.float32),
        ],
    )
    def k(u_hbm, src_hbm, dst_hbm, out_hbm, srcb, dstb, rows0, rows1, sem0, sem1, acc):
        cid = lax.axis_index("c")
        sid = lax.axis_index("s")

        @pl.loop(0, CH)
        def _(r):
            for j in range(HW // 16):
                rows0[r, pl.ds(j * 16, 16)] = jnp.zeros((16,), jnp.float32)

        for j in range(pl.cdiv(n_rchunks, NS)):
            kk = sid + NS * j
            @pl.when(kk < n_rchunks)
            def _():
                pltpu.sync_copy(rows0, acc.at[pl.ds(kk * CH, CH)])
        plsc.subcore_barrier()

        pltpu.sync_copy(src_hbm.at[sid], srcb)
        pltpu.sync_copy(dst_hbm.at[sid], dstb)

        def gather(c, buf, sem):
            pltpu.async_copy(u_hbm.at[cid].at[srcb.at[c]], buf, sem)

        def gwait(buf, sem):
            pltpu.make_async_copy(u_hbm.at[cid].at[srcb.at[0]], buf, sem).wait()

        def scat(c, buf):
            pltpu.sync_copy(buf, acc.at[dstb.at[c]], add=True)

        gather(0, rows0, sem0)

        @pl.loop(0, n_chunks // 2 - 1)
        def _(i):
            c0 = 2 * i
            gather(c0 + 1, rows1, sem1)
            gwait(rows0, sem0)
            scat(c0, rows0)
            gather(c0 + 2, rows0, sem0)
            gwait(rows1, sem1)
            scat(c0 + 1, rows1)

        gather(n_chunks - 1, rows1, sem1)
        gwait(rows0, sem0)
        scat(n_chunks - 2, rows0)
        gwait(rows1, sem1)
        scat(n_chunks - 1, rows1)
        plsc.subcore_barrier()

        for j in range(pl.cdiv(n_rchunks, NS)):
            kk = sid + NS * j
            @pl.when(kk < n_rchunks)
            def _():
                pltpu.sync_copy(acc.at[pl.ds(kk * CH, CH)],
                                out_hbm.at[cid, pl.ds(kk * CH, CH)])

    return k


@functools.lru_cache(maxsize=None)
def _degree_kernel(n_nodes, n_edges):
    per_w = n_edges // NW
    assert per_w * NW == n_edges
    n_chunks = per_w // CH
    assert n_chunks * CH == per_w
    n_rchunks = n_nodes // CH
    assert n_rchunks * CH == n_nodes

    @functools.partial(
        pl.kernel,
        out_type=jax.ShapeDtypeStruct((NC, n_nodes, DEGW), jnp.float32),
        mesh=_sc_mesh(),
        compiler_params=pltpu.CompilerParams(use_tc_tiling_on_sc=False),
        scratch_types=[
            pltpu.VMEM((n_chunks, CH), jnp.int32),
            pltpu.VMEM((CH, DEGW), jnp.float32),
            pltpu.VMEM((CH, DEGW), jnp.float32),
            pltpu.SemaphoreType.DMA,
            pltpu.VMEM_SHARED((n_nodes, DEGW), jnp.float32),
        ],
    )
    def k(dst_hbm, out_hbm, dstb, zeros_v, ones_v, sem, acc):
        cid = lax.axis_index("c")
        sid = lax.axis_index("s")
        wid = cid * NS + sid

        @pl.loop(0, CH)
        def _(r):
            zeros_v[r, :] = jnp.zeros((DEGW,), jnp.float32)
            ones_v[r, :] = jnp.ones((DEGW,), jnp.float32)

        for j in range(pl.cdiv(n_rchunks, NS)):
            kk = sid + NS * j
            @pl.when(kk < n_rchunks)
            def _():
                pltpu.sync_copy(zeros_v, acc.at[pl.ds(kk * CH, CH)])
        plsc.subcore_barrier()

        pltpu.sync_copy(dst_hbm.at[wid], dstb)

        @pl.loop(0, n_chunks)
        def _(c):
            pltpu.async_copy(ones_v, acc.at[dstb.at[c]], sem, add=True)

        @pl.loop(0, n_chunks)
        def _(c):
            pltpu.make_async_copy(ones_v, acc.at[dstb.at[0]], sem).wait()

        plsc.subcore_barrier()

        for j in range(pl.cdiv(n_rchunks, NS)):
            kk = sid + NS * j
            @pl.when(kk < n_rchunks)
            def _():
                pltpu.sync_copy(acc.at[pl.ds(kk * CH, CH)],
                                out_hbm.at[cid, pl.ds(kk * CH, CH)])

    return k


def _tc_first(x, w0, degp):
    n, d = x.shape

    def body(x_ref, w_ref, degp_ref, dis_ref, u_ref):
        deg = degp_ref[0, :, 0:1] + degp_ref[1, :, 0:1] + 1.0
        dis = lax.rsqrt(deg)
        dis_ref[...] = dis
        u = dis * jnp.dot(x_ref[...], w_ref[...],
                          preferred_element_type=jnp.float32)
        u_ref[0] = u[:, :HW]
        u_ref[1] = u[:, HW:]

    return pl.pallas_call(
        body,
        out_shape=(
            jax.ShapeDtypeStruct((n, 1), jnp.float32),
            jax.ShapeDtypeStruct((NC, n, HW), jnp.float32),
        ),
    )(x, w0, degp)


def _tc_mid(p, u, dis, b, w):
    n = u.shape[1]

    def body(p_ref, u_ref, dis_ref, b_ref, w_ref, un_ref):
        s2 = p_ref[...] + u_ref[...]
        s = jnp.concatenate([s2[0], s2[1]], axis=1)
        h = jnp.maximum(dis_ref[...] * s + b_ref[...], 0.0)
        un = dis_ref[...] * jnp.dot(h, w_ref[...],
                                    preferred_element_type=jnp.float32)
        un_ref[0] = un[:, :HW]
        un_ref[1] = un[:, HW:]

    return pl.pallas_call(
        body,
        out_shape=jax.ShapeDtypeStruct((NC, n, HW), jnp.float32),
    )(p, u, dis, b.reshape(1, -1), w)


def _tc_last(p, u, dis, b, batch, lin_w, lin_b):
    n = u.shape[1]
    ncls = lin_w.shape[1]

    def body(p_ref, u_ref, dis_ref, b_ref, batch_ref, lw_ref, lb_ref, out_ref):
        s2 = p_ref[...] + u_ref[...]
        s = jnp.concatenate([s2[0], s2[1]], axis=1)
        h = jnp.maximum(dis_ref[...] * s + b_ref[...], 0.0)
        oh = (batch_ref[...] ==
              lax.broadcasted_iota(jnp.int32, (n, NUM_GRAPHS), 1)
              ).astype(jnp.float32)
        sums = lax.dot_general(oh, h, (((0,), (0,)), ((), ())),
                               preferred_element_type=jnp.float32)
        counts = jnp.sum(oh, axis=0)
        pooled = sums / jnp.maximum(counts, 1.0)[:, None]
        out_ref[...] = jnp.dot(pooled, lw_ref[...],
                               preferred_element_type=jnp.float32) + lb_ref[...]

    return pl.pallas_call(
        body,
        out_shape=jax.ShapeDtypeStruct((NUM_GRAPHS, ncls), jnp.float32),
    )(p, u, dis, b.reshape(1, -1), batch.reshape(-1, 1), lin_w, lin_b.reshape(1, -1))


def kernel(x, edge_index, batch, W0, b0, W1, b1, W2, b2, W3, b3, W4, b4, lin_W, lin_b):
    n, d = x.shape
    e = edge_index.shape[1]
    src16 = edge_index[0].reshape(NS, -1, CH)
    dst16 = edge_index[1].reshape(NS, -1, CH)
    dst32 = edge_index[1].reshape(NW, -1, CH)

    degp = _degree_kernel(n, e)(dst32)
    dis, u = _tc_first(x, W0, degp)

    scatter = _edge_scatter_kernel(n, e)
    for w_next, b_cur in ((W1, b0), (W2, b1), (W3, b2), (W4, b3)):
        p = scatter(u, src16, dst16)
        u = _tc_mid(p, u, dis, b_cur, w_next)

    p = scatter(u, src16, dst16)
    return _tc_last(p, u, dis, b4, batch, lin_W, lin_b)
